# Initial kernel scaffold; baseline (speedup 1.0000x reference)
import jax
import jax.numpy as jnp
from jax.experimental import pallas as pl


def kernel(x, table):
    raise NotImplementedError("write your pallas kernel here")


# speedup vs baseline: 3.4931x; 3.4931x over previous
import numpy as np
import jax
import jax.numpy as jnp
from jax.experimental import pallas as pl
from jax.experimental.pallas import tpu as pltpu

VOCAB = 64
EMBED = 512
SEQ = 512
BATCH = 128
SCALE = float(np.sqrt(EMBED))


def _pos_encoding(length, depth):
    half = depth // 2
    positions = np.arange(length)[:, np.newaxis]
    depths = np.arange(half)[np.newaxis, :] / half
    angle_rates = 1 / 10000 ** depths
    angle_rads = positions * angle_rates
    return np.concatenate(
        [np.sin(angle_rads), np.cos(angle_rads)], axis=-1
    ).astype(np.float32)


def _tc_body(x_ref, table_ref, pos_ref, out_ref):
    x = x_ref[0, 0, :]
    onehot = jnp.where(
        x[:, None] == jax.lax.broadcasted_iota(jnp.int32, (SEQ, VOCAB), 1),
        SCALE,
        0.0,
    ).astype(jnp.float32)
    emb = jnp.dot(onehot, table_ref[...], preferred_element_type=jnp.float32)
    out_ref[0, :, :] = emb + pos_ref[...]


def kernel(x, table):
    pos = jnp.asarray(_pos_encoding(SEQ, EMBED))
    x3 = x.astype(jnp.int32).reshape(BATCH, 1, SEQ)
    out = pl.pallas_call(
        _tc_body,
        grid=(BATCH,),
        in_specs=[
            pl.BlockSpec((1, 1, SEQ), lambda b: (b, 0, 0)),
            pl.BlockSpec((VOCAB, EMBED), lambda b: (0, 0)),
            pl.BlockSpec((SEQ, EMBED), lambda b: (0, 0)),
        ],
        out_specs=pl.BlockSpec((1, SEQ, EMBED), lambda b: (b, 0, 0)),
        out_shape=jax.ShapeDtypeStruct((BATCH, SEQ, EMBED), jnp.float32),
    )(x3, table, pos)
    return out


# speedup vs baseline: 7.0168x; 2.0087x over previous
import numpy as np
import jax
import jax.numpy as jnp
from jax.experimental import pallas as pl
from jax.experimental.pallas import tpu as pltpu

VOCAB = 64
EMBED = 512
SEQ = 512
BATCH = 128
SCALE = float(np.sqrt(EMBED))


def _pos_encoding(length, depth):
    half = depth // 2
    positions = np.arange(length)[:, np.newaxis]
    depths = np.arange(half)[np.newaxis, :] / half
    angle_rates = 1 / 10000 ** depths
    angle_rads = positions * angle_rates
    return np.concatenate(
        [np.sin(angle_rads), np.cos(angle_rads)], axis=-1
    ).astype(np.float32)


BB = 4


def _tc_body(x_ref, table_ref, pos_ref, out_ref):
    x = x_ref[0, 0, :]
    onehot = jnp.where(
        x[:, None]
        == jax.lax.broadcasted_iota(jnp.int32, (BB * SEQ, VOCAB), 1),
        SCALE,
        0.0,
    ).astype(jnp.float32)
    emb = jnp.dot(onehot, table_ref[...], preferred_element_type=jnp.float32)
    pos = pos_ref[...]
    for i in range(BB):
        out_ref[i * SEQ : (i + 1) * SEQ, :] = (
            emb[i * SEQ : (i + 1) * SEQ, :] + pos
        )


def kernel(x, table):
    pos = jnp.asarray(_pos_encoding(SEQ, EMBED))
    x3 = x.astype(jnp.int32).reshape(BATCH // BB, 1, BB * SEQ)
    out = pl.pallas_call(
        _tc_body,
        grid=(BATCH // BB,),
        in_specs=[
            pl.BlockSpec((1, 1, BB * SEQ), lambda b: (b, 0, 0)),
            pl.BlockSpec((VOCAB, EMBED), lambda b: (0, 0)),
            pl.BlockSpec((SEQ, EMBED), lambda b: (0, 0)),
        ],
        out_specs=pl.BlockSpec((BB * SEQ, EMBED), lambda b: (b, 0)),
        out_shape=jax.ShapeDtypeStruct((BATCH * SEQ, EMBED), jnp.float32),
    )(x3, table, pos)
    return out.reshape(BATCH, SEQ, EMBED)


# speedup vs baseline: 7.2541x; 1.0338x over previous
import numpy as np
import jax
import jax.numpy as jnp
from jax.experimental import pallas as pl
from jax.experimental.pallas import tpu as pltpu

VOCAB = 64
EMBED = 512
SEQ = 512
BATCH = 128
SCALE = float(np.sqrt(EMBED))


def _pos_encoding(length, depth):
    half = depth // 2
    positions = np.arange(length)[:, np.newaxis]
    depths = np.arange(half)[np.newaxis, :] / half
    angle_rates = 1 / 10000 ** depths
    angle_rads = positions * angle_rates
    return np.concatenate(
        [np.sin(angle_rads), np.cos(angle_rads)], axis=-1
    ).astype(np.float32)


BB = 8


def _tc_body(x_ref, table_ref, pos_ref, out_ref):
    x = x_ref[0, 0, :]
    onehot = jnp.where(
        x[:, None]
        == jax.lax.broadcasted_iota(jnp.int32, (BB * SEQ, VOCAB), 1),
        SCALE,
        0.0,
    ).astype(jnp.float32)
    emb = jnp.dot(onehot, table_ref[...], preferred_element_type=jnp.float32)
    pos = pos_ref[...]
    for i in range(BB):
        out_ref[i * SEQ : (i + 1) * SEQ, :] = (
            emb[i * SEQ : (i + 1) * SEQ, :] + pos
        )


def kernel(x, table):
    pos = jnp.asarray(_pos_encoding(SEQ, EMBED))
    x3 = x.astype(jnp.int32).reshape(BATCH // BB, 1, BB * SEQ)
    out = pl.pallas_call(
        _tc_body,
        grid=(BATCH // BB,),
        in_specs=[
            pl.BlockSpec((1, 1, BB * SEQ), lambda b: (b, 0, 0)),
            pl.BlockSpec((VOCAB, EMBED), lambda b: (0, 0)),
            pl.BlockSpec((SEQ, EMBED), lambda b: (0, 0)),
        ],
        out_specs=pl.BlockSpec((BB * SEQ, EMBED), lambda b: (b, 0)),
        out_shape=jax.ShapeDtypeStruct((BATCH * SEQ, EMBED), jnp.float32),
    )(x3, table, pos)
    return out.reshape(BATCH, SEQ, EMBED)
